# Initial kernel scaffold; baseline (speedup 1.0000x reference)
import jax
import jax.numpy as jnp
from jax.experimental import pallas as pl


def kernel(X, edge_index, edge_weight, amp_weight, gate_w, gate_b, sage_w, sage_b):
    raise NotImplementedError("write your pallas kernel here")


# speedup vs baseline: 7.0241x; 7.0241x over previous
import functools

import jax
import jax.numpy as jnp
from jax import lax
from jax.experimental import pallas as pl
from jax.experimental.pallas import tpu as pltpu
from jax.experimental.pallas import tpu_sc as plsc

N = 10000
E = 320000
D = 128
H = D // 2
OUT = 128

NW = 32
EPT = E // NW
K = 80
NCH = EPT // K
OWN = 640
ZR = 80


def _splat(vec16, lane):
    idx = jnp.full((16, 1), lane, dtype=jnp.int32)
    return lax.gather(
        vec16, idx,
        lax.GatherDimensionNumbers(offset_dims=(), collapsed_slice_dims=(0,),
                                   start_index_map=(0,)),
        (1,), mode=lax.GatherScatterMode.PROMISE_IN_BOUNDS)


def _gate_body(x_ref, w_ref, o_ref):
    o_ref[...] = jnp.dot(x_ref[...], w_ref[...],
                         preferred_element_type=jnp.float32)


def _gate_proj(x2, gw2):
    blk = 2000
    return pl.pallas_call(
        _gate_body,
        grid=(N // blk,),
        in_specs=[
            pl.BlockSpec((blk, D), lambda i: (i, 0)),
            pl.BlockSpec((D, 2), lambda i: (0, 0)),
        ],
        out_specs=pl.BlockSpec((blk, 2), lambda i: (i, 0)),
        out_shape=jax.ShapeDtypeStruct((N, 2), jnp.float32),
    )(x2, gw2)


def _sc_body(xlo_hbm, xhi_hbm, src_hbm, dst_hbm, ew_hbm, gi_hbm, gj_hbm,
             gc_hbm, s_out, deg_out,
             src_v, dst_v, ew_v, coeff_v, gi_v, gj_v, gc_v,
             rows_v, srcc_v, dstc_v, ones_v, zero_v, degz_v,
             S_sh, deg_sh, sem):
    cid = lax.axis_index("c")
    sid = lax.axis_index("s")
    wid = sid * 2 + cid
    ebase = wid * EPT

    pltpu.sync_copy(src_hbm.at[pl.ds(ebase, EPT)], src_v)
    pltpu.sync_copy(dst_hbm.at[pl.ds(ebase, EPT)], dst_v)
    pltpu.sync_copy(ew_hbm.at[pl.ds(ebase, EPT)], ew_v)
    pltpu.sync_copy(gi_hbm, gi_v)
    pltpu.sync_copy(gj_hbm, gj_v)
    pltpu.sync_copy(gc_hbm, gc_v)

    z16 = jnp.zeros((16,), jnp.float32)
    p16 = jnp.where(lax.iota(jnp.int32, 16) == 0,
                    jnp.float32(1.0), jnp.float32(0.0))

    def _zv(i, c):
        zero_v[i // 4, pl.ds((i % 4) * 16, 16)] = z16
        return c
    lax.fori_loop(0, ZR * (H // 16), _zv, 0)

    def _dz(r, c):
        degz_v[r, pl.ds(0, 16)] = z16
        ones_v[r, pl.ds(0, 16)] = p16
        return c
    lax.fori_loop(0, ZR, _dz, 0)

    gcv = gc_v[pl.ds(0, 16)]
    gwe = _splat(gcv, 0)
    gb = _splat(gcv, 1)

    def _coef(i, c):
        off = i * 16
        s16 = src_v[pl.ds(off, 16)]
        d16 = dst_v[pl.ds(off, 16)]
        gi = plsc.load_gather(gi_v, [d16])
        gj = plsc.load_gather(gj_v, [s16])
        w16 = ew_v[pl.ds(off, 16)]
        t = gi + gj + w16 * gwe + gb
        lamb = 1.0 / (1.0 + jnp.exp(-t))
        coeff_v[pl.ds(off, 16)] = w16 * lamb
        return c
    lax.fori_loop(0, EPT // 16, _coef, 0)

    row0 = sid * OWN
    nzc = jnp.where(sid == 15, (N - 15 * OWN) // ZR, OWN // ZR)

    for p in range(2):
        x_hbm = xlo_hbm if p == 0 else xhi_hbm

        def _zs(j, c):
            pltpu.sync_copy(zero_v, S_sh.at[pl.ds(row0 + j * ZR, ZR)])
            if p == 0:
                pltpu.sync_copy(degz_v, deg_sh.at[pl.ds(row0 + j * ZR, ZR)])
            return c
        lax.fori_loop(0, nzc, _zs, 0)

        plsc.subcore_barrier()

        def _chunk(ch, c):
            eoff = ch * K
            for g in range(K // 16):
                srcc_v[0, pl.ds(g * 16, 16)] = src_v[pl.ds(eoff + g * 16, 16)]
                dstc_v[0, pl.ds(g * 16, 16)] = dst_v[pl.ds(eoff + g * 16, 16)]
            pltpu.async_copy(x_hbm.at[srcc_v.at[0]], rows_v, sem).wait()
            for g in range(K // 16):
                c16 = coeff_v[pl.ds(eoff + g * 16, 16)]
                for l in range(16):
                    e = g * 16 + l
                    sp = _splat(c16, l)
                    for col in range(H // 16):
                        cs = col * 16
                        rows_v[e, pl.ds(cs, 16)] = (
                            rows_v[e, pl.ds(cs, 16)] * sp)
            pltpu.sync_copy(rows_v, S_sh.at[dstc_v.at[0]], add=True)
            if p == 0:
                pltpu.sync_copy(ones_v, deg_sh.at[dstc_v.at[0]], add=True)
            return c
        lax.fori_loop(0, NCH, _chunk, 0)

        plsc.subcore_barrier()

        def _wo(j, c):
            r = row0 + j * ZR
            pltpu.sync_copy(S_sh.at[pl.ds(r, ZR)],
                            s_out.at[cid, p, pl.ds(r, ZR)])
            if p == 0:
                pltpu.sync_copy(deg_sh.at[pl.ds(r, ZR)],
                                deg_out.at[cid, pl.ds(r, ZR)])
            return c
        lax.fori_loop(0, nzc, _wo, 0)

        if p == 0:
            plsc.subcore_barrier()


def _sc_aggregate(xlo, xhi, src, dst, ew, gi, gj, gc):
    mesh = plsc.VectorSubcoreMesh(core_axis_name="c", subcore_axis_name="s")
    f = functools.partial(
        pl.kernel,
        mesh=mesh,
        compiler_params=pltpu.CompilerParams(needs_layout_passes=False,
                                             use_tc_tiling_on_sc=False),
        out_type=[
            jax.ShapeDtypeStruct((2, 2, N, H), jnp.float32),
            jax.ShapeDtypeStruct((2, N, 16), jnp.float32),
        ],
        scratch_types=[
            pltpu.VMEM((EPT,), jnp.int32),
            pltpu.VMEM((EPT,), jnp.int32),
            pltpu.VMEM((EPT,), jnp.float32),
            pltpu.VMEM((EPT,), jnp.float32),
            pltpu.VMEM((N,), jnp.float32),
            pltpu.VMEM((N,), jnp.float32),
            pltpu.VMEM((16,), jnp.float32),
            pltpu.VMEM((K, H), jnp.float32),
            pltpu.VMEM((1, K), jnp.int32),
            pltpu.VMEM((1, K), jnp.int32),
            pltpu.VMEM((ZR, 16), jnp.float32),
            pltpu.VMEM((ZR, H), jnp.float32),
            pltpu.VMEM((ZR, 16), jnp.float32),
            pltpu.VMEM_SHARED((N, H), jnp.float32),
            pltpu.VMEM_SHARED((N, 16), jnp.float32),
            pltpu.SemaphoreType.DMA,
        ],
    )(_sc_body)
    return f(xlo, xhi, src, dst, ew, gi, gj, gc)


def _final_body(x_ref, s_ref, d_ref, amp_ref, w_ref, b_ref, o_ref):
    s_lo = s_ref[0, 0] + s_ref[1, 0]
    s_hi = s_ref[0, 1] + s_ref[1, 1]
    s = jnp.concatenate([s_lo, s_hi], axis=-1)
    dg = jnp.maximum(d_ref[0, :, 0:1] + d_ref[1, :, 0:1], 1.0)
    aggr = s * amp_ref[...] / dg
    o_ref[...] = (jnp.dot(x_ref[...], w_ref[0:D, :],
                          preferred_element_type=jnp.float32)
                  + jnp.dot(aggr, w_ref[D:2 * D, :],
                            preferred_element_type=jnp.float32)
                  + b_ref[...])


def _final(x2, s_parts, deg_parts, amp_weight, sage_w, sage_b2):
    blk = 400
    return pl.pallas_call(
        _final_body,
        grid=(N // blk,),
        in_specs=[
            pl.BlockSpec((blk, D), lambda i: (i, 0)),
            pl.BlockSpec((2, 2, blk, H), lambda i: (0, 0, i, 0)),
            pl.BlockSpec((2, blk, 16), lambda i: (0, i, 0)),
            pl.BlockSpec((1, D), lambda i: (0, 0)),
            pl.BlockSpec((2 * D, OUT), lambda i: (0, 0)),
            pl.BlockSpec((1, OUT), lambda i: (0, 0)),
        ],
        out_specs=pl.BlockSpec((blk, OUT), lambda i: (i, 0)),
        out_shape=jax.ShapeDtypeStruct((N, OUT), jnp.float32),
    )(x2, s_parts, deg_parts, amp_weight, sage_w, sage_b2)


def kernel(X, edge_index, edge_weight, amp_weight, gate_w, gate_b, sage_w,
           sage_b):
    x2 = X[0]
    src = edge_index[0]
    dst = edge_index[1]
    gw2 = jnp.stack([gate_w[:D, 0], gate_w[D:2 * D, 0]], axis=1)
    gc = jnp.zeros((16,), jnp.float32)
    gc = gc.at[0].set(gate_w[2 * D, 0]).at[1].set(gate_b[0])

    g2 = _gate_proj(x2, gw2)
    gi = g2[:, 0]
    gj = g2[:, 1]

    xlo = x2[:, :H]
    xhi = x2[:, H:]
    s_parts, deg_parts = _sc_aggregate(xlo, xhi, src, dst, edge_weight,
                                       gi, gj, gc)

    out2 = _final(x2, s_parts, deg_parts, amp_weight, sage_w,
                  sage_b.reshape(1, OUT))
    return out2[None]


# speedup vs baseline: 10.4322x; 1.4852x over previous
import functools

import jax
import jax.numpy as jnp
from jax import lax
from jax.experimental import pallas as pl
from jax.experimental.pallas import tpu as pltpu
from jax.experimental.pallas import tpu_sc as plsc

N = 10000
E = 320000
D = 128
H = D // 2
OUT = 128

NW = 32
EPT = E // NW
K = 80
NCH = EPT // K
OWN = 640
ZR = 80


def _splat(vec16, lane):
    idx = jnp.full((16, 1), lane, dtype=jnp.int32)
    return lax.gather(
        vec16, idx,
        lax.GatherDimensionNumbers(offset_dims=(), collapsed_slice_dims=(0,),
                                   start_index_map=(0,)),
        (1,), mode=lax.GatherScatterMode.PROMISE_IN_BOUNDS)


def _gate_body(x_ref, w_ref, o_ref):
    o_ref[...] = jnp.dot(x_ref[...], w_ref[...],
                         preferred_element_type=jnp.float32)


def _gate_proj(x2, gw2):
    blk = 2000
    return pl.pallas_call(
        _gate_body,
        grid=(N // blk,),
        in_specs=[
            pl.BlockSpec((blk, D), lambda i: (i, 0)),
            pl.BlockSpec((D, 2), lambda i: (0, 0)),
        ],
        out_specs=pl.BlockSpec((blk, 2), lambda i: (i, 0)),
        out_shape=jax.ShapeDtypeStruct((N, 2), jnp.float32),
    )(x2, gw2)


def _sc_body(xlo_hbm, xhi_hbm, src_hbm, dst_hbm, ew_hbm, gi_hbm, gj_hbm,
             gc_hbm, s_out, deg_out,
             src_v, dst_v, ew_v, coeff_v, gi_v, gj_v, gc_v,
             rows0_v, rows1_v, srcc_v, dstc_v, ones_v, zero_v, degz_v,
             S_sh, deg_sh, sem0, sem1):
    cid = lax.axis_index("c")
    sid = lax.axis_index("s")
    wid = sid * 2 + cid
    ebase = wid * EPT

    pltpu.sync_copy(src_hbm.at[pl.ds(ebase, EPT)], src_v)
    pltpu.sync_copy(dst_hbm.at[pl.ds(ebase, EPT)], dst_v)
    pltpu.sync_copy(ew_hbm.at[pl.ds(ebase, EPT)], ew_v)
    pltpu.sync_copy(gi_hbm, gi_v)
    pltpu.sync_copy(gj_hbm, gj_v)
    pltpu.sync_copy(gc_hbm, gc_v)

    z16 = jnp.zeros((16,), jnp.float32)
    p16 = jnp.where(lax.iota(jnp.int32, 16) == 0,
                    jnp.float32(1.0), jnp.float32(0.0))

    def _zv(i, c):
        zero_v[i // 4, pl.ds((i % 4) * 16, 16)] = z16
        return c
    lax.fori_loop(0, ZR * (H // 16), _zv, 0)

    def _dz(r, c):
        degz_v[r, pl.ds(0, 16)] = z16
        ones_v[r, pl.ds(0, 16)] = p16
        return c
    lax.fori_loop(0, ZR, _dz, 0)

    gcv = gc_v[pl.ds(0, 16)]
    gwe = _splat(gcv, 0)
    gb = _splat(gcv, 1)

    def _coef(i, c):
        off = i * 16
        s16 = src_v[pl.ds(off, 16)]
        d16 = dst_v[pl.ds(off, 16)]
        gi = plsc.load_gather(gi_v, [d16])
        gj = plsc.load_gather(gj_v, [s16])
        w16 = ew_v[pl.ds(off, 16)]
        t = gi + gj + w16 * gwe + gb
        lamb = 1.0 / (1.0 + jnp.exp(-t))
        coeff_v[pl.ds(off, 16)] = w16 * lamb
        return c
    lax.fori_loop(0, EPT // 16, _coef, 0)

    row0 = sid * OWN
    nzc = jnp.where(sid == 15, (N - 15 * OWN) // ZR, OWN // ZR)

    for p in range(2):
        x_hbm = xlo_hbm if p == 0 else xhi_hbm

        def _zs(j, c):
            pltpu.sync_copy(zero_v, S_sh.at[pl.ds(row0 + j * ZR, ZR)])
            if p == 0:
                pltpu.sync_copy(degz_v, deg_sh.at[pl.ds(row0 + j * ZR, ZR)])
            return c
        lax.fori_loop(0, nzc, _zs, 0)

        plsc.subcore_barrier()

        def _fill(slot, ch):
            eoff = ch * K
            for g in range(K // 16):
                srcc_v[slot, pl.ds(g * 16, 16)] = (
                    src_v[pl.ds(eoff + g * 16, 16)])
                dstc_v[slot, pl.ds(g * 16, 16)] = (
                    dst_v[pl.ds(eoff + g * 16, 16)])

        def _scale(rows, ch):
            eoff = ch * K
            for g in range(K // 16):
                c16 = coeff_v[pl.ds(eoff + g * 16, 16)]
                for l in range(16):
                    e = g * 16 + l
                    sp = _splat(c16, l)
                    for col in range(H // 16):
                        cs = col * 16
                        rows[e, pl.ds(cs, 16)] = rows[e, pl.ds(cs, 16)] * sp

        def _scatter(rows, slot):
            pltpu.sync_copy(rows, S_sh.at[dstc_v.at[slot]], add=True)
            if p == 0:
                pltpu.sync_copy(ones_v, deg_sh.at[dstc_v.at[slot]], add=True)

        _fill(0, 0)
        pltpu.async_copy(x_hbm.at[srcc_v.at[0]], rows0_v, sem0)

        def _pair(i, c):
            c0 = 2 * i
            _fill(1, c0 + 1)
            pltpu.async_copy(x_hbm.at[srcc_v.at[1]], rows1_v, sem1)
            pltpu.make_async_copy(x_hbm.at[srcc_v.at[0]], rows0_v,
                                  sem0).wait()
            _scale(rows0_v, c0)
            _scatter(rows0_v, 0)
            _fill(0, c0 + 2)
            pltpu.async_copy(x_hbm.at[srcc_v.at[0]], rows0_v, sem0)
            pltpu.make_async_copy(x_hbm.at[srcc_v.at[1]], rows1_v,
                                  sem1).wait()
            _scale(rows1_v, c0 + 1)
            _scatter(rows1_v, 1)
            return c
        lax.fori_loop(0, (NCH - 1) // 2, _pair, 0)

        pltpu.make_async_copy(x_hbm.at[srcc_v.at[0]], rows0_v, sem0).wait()
        _scale(rows0_v, NCH - 1)
        _scatter(rows0_v, 0)

        plsc.subcore_barrier()

        def _wo(j, c):
            r = row0 + j * ZR
            pltpu.sync_copy(S_sh.at[pl.ds(r, ZR)],
                            s_out.at[cid, p, pl.ds(r, ZR)])
            if p == 0:
                pltpu.sync_copy(deg_sh.at[pl.ds(r, ZR)],
                                deg_out.at[cid, pl.ds(r, ZR)])
            return c
        lax.fori_loop(0, nzc, _wo, 0)

        if p == 0:
            plsc.subcore_barrier()


def _sc_aggregate(xlo, xhi, src, dst, ew, gi, gj, gc):
    mesh = plsc.VectorSubcoreMesh(core_axis_name="c", subcore_axis_name="s")
    f = functools.partial(
        pl.kernel,
        mesh=mesh,
        compiler_params=pltpu.CompilerParams(needs_layout_passes=False,
                                             use_tc_tiling_on_sc=False),
        out_type=[
            jax.ShapeDtypeStruct((2, 2, N, H), jnp.float32),
            jax.ShapeDtypeStruct((2, N, 16), jnp.float32),
        ],
        scratch_types=[
            pltpu.VMEM((EPT,), jnp.int32),
            pltpu.VMEM((EPT,), jnp.int32),
            pltpu.VMEM((EPT,), jnp.float32),
            pltpu.VMEM((EPT,), jnp.float32),
            pltpu.VMEM((N,), jnp.float32),
            pltpu.VMEM((N,), jnp.float32),
            pltpu.VMEM((16,), jnp.float32),
            pltpu.VMEM((K, H), jnp.float32),
            pltpu.VMEM((K, H), jnp.float32),
            pltpu.VMEM((2, K), jnp.int32),
            pltpu.VMEM((2, K), jnp.int32),
            pltpu.VMEM((ZR, 16), jnp.float32),
            pltpu.VMEM((ZR, H), jnp.float32),
            pltpu.VMEM((ZR, 16), jnp.float32),
            pltpu.VMEM_SHARED((N, H), jnp.float32),
            pltpu.VMEM_SHARED((N, 16), jnp.float32),
            pltpu.SemaphoreType.DMA,
            pltpu.SemaphoreType.DMA,
        ],
    )(_sc_body)
    return f(xlo, xhi, src, dst, ew, gi, gj, gc)


def _final_body(x_ref, s_ref, d_ref, amp_ref, w_ref, b_ref, o_ref):
    s_lo = s_ref[0, 0] + s_ref[1, 0]
    s_hi = s_ref[0, 1] + s_ref[1, 1]
    s = jnp.concatenate([s_lo, s_hi], axis=-1)
    dg = jnp.maximum(d_ref[0, :, 0:1] + d_ref[1, :, 0:1], 1.0)
    aggr = s * amp_ref[...] / dg
    o_ref[...] = (jnp.dot(x_ref[...], w_ref[0:D, :],
                          preferred_element_type=jnp.float32)
                  + jnp.dot(aggr, w_ref[D:2 * D, :],
                            preferred_element_type=jnp.float32)
                  + b_ref[...])


def _final(x2, s_parts, deg_parts, amp_weight, sage_w, sage_b2):
    blk = 400
    return pl.pallas_call(
        _final_body,
        grid=(N // blk,),
        in_specs=[
            pl.BlockSpec((blk, D), lambda i: (i, 0)),
            pl.BlockSpec((2, 2, blk, H), lambda i: (0, 0, i, 0)),
            pl.BlockSpec((2, blk, 16), lambda i: (0, i, 0)),
            pl.BlockSpec((1, D), lambda i: (0, 0)),
            pl.BlockSpec((2 * D, OUT), lambda i: (0, 0)),
            pl.BlockSpec((1, OUT), lambda i: (0, 0)),
        ],
        out_specs=pl.BlockSpec((blk, OUT), lambda i: (i, 0)),
        out_shape=jax.ShapeDtypeStruct((N, OUT), jnp.float32),
    )(x2, s_parts, deg_parts, amp_weight, sage_w, sage_b2)


def kernel(X, edge_index, edge_weight, amp_weight, gate_w, gate_b, sage_w,
           sage_b):
    x2 = X[0]
    src = edge_index[0]
    dst = edge_index[1]
    gw2 = jnp.stack([gate_w[:D, 0], gate_w[D:2 * D, 0]], axis=1)
    gc = jnp.zeros((16,), jnp.float32)
    gc = gc.at[0].set(gate_w[2 * D, 0]).at[1].set(gate_b[0])

    g2 = _gate_proj(x2, gw2)
    gi = g2[:, 0]
    gj = g2[:, 1]

    xlo = x2[:, :H]
    xhi = x2[:, H:]
    s_parts, deg_parts = _sc_aggregate(xlo, xhi, src, dst, edge_weight,
                                       gi, gj, gc)

    out2 = _final(x2, s_parts, deg_parts, amp_weight, sage_w,
                  sage_b.reshape(1, OUT))
    return out2[None]
